# baseline (speedup 1.0000x reference)
import functools

import jax
import jax.numpy as jnp
from jax import lax
from jax.experimental import pallas as pl
from jax.experimental.pallas import tpu as pltpu
from jax.experimental.pallas import tpu_sc as plsc

VOCAB = 1000
BATCH = 16384
HIST = 200
D = 32
B = BATCH * HIST
NC = 2
NS = 16
NW = NC * NS
TOK_PER_W = B // NW
CHUNK = 1024
NITER = TOK_PER_W // CHUNK
NGRP = CHUNK // 16
NBUF = 2
NBODY = NITER // NBUF

_mesh = plsc.VectorSubcoreMesh(core_axis_name="c", subcore_axis_name="s")


@functools.partial(
    pl.kernel,
    mesh=_mesh,
    compiler_params=pltpu.CompilerParams(needs_layout_passes=False),
    out_type=jax.ShapeDtypeStruct((B * D,), jnp.float32),
    scratch_types=[
        pltpu.VMEM((VOCAB * D,), jnp.float32),
        [pltpu.VMEM((CHUNK,), jnp.int32)] * NBUF,
        [pltpu.VMEM((CHUNK * D,), jnp.float32)] * NBUF,
        [pltpu.SemaphoreType.DMA] * NBUF,
        [pltpu.SemaphoreType.DMA] * NBUF,
    ],
)
def _emb_lookup(idx_hbm, w_hbm, out_hbm, w_loc, idx_v, rows_v, ssems, isems):
    wid = lax.axis_index("s") * NC + lax.axis_index("c")
    base = wid * TOK_PER_W

    pltpu.sync_copy(w_hbm, w_loc)

    lane32 = lax.iota(jnp.int32, 16) * D

    def body(s, carry):
        for k in range(NBUF):
            i = s * NBUF + k
            t0 = base + i * CHUNK
            idx_b = idx_v[k]
            rows_b = rows_v[k]

            @pl.when(s > 0)
            def _drain_prev():
                pltpu.make_async_copy(
                    rows_b, out_hbm.at[pl.ds(t0 * D, CHUNK * D)], ssems[k]
                ).wait()
                pltpu.make_async_copy(
                    idx_hbm.at[pl.ds(t0, CHUNK)], idx_b, isems[k]
                ).wait()

            @pl.when(s == 0)
            def _prime_idx():
                pltpu.sync_copy(idx_hbm.at[pl.ds(t0, CHUNK)], idx_b)

            def group(g, c):
                tok = idx_b[pl.ds(g * 16, 16)]
                gbase = tok * D
                obase = lane32 + g * (16 * D)
                for d in range(D):
                    vals = plsc.load_gather(w_loc, [gbase + d])
                    plsc.store_scatter(rows_b, [obase + d], vals)
                return c

            lax.fori_loop(0, NGRP, group, 0)

            @pl.when(s < NBODY - 1)
            def _prefetch_idx():
                t0n = t0 + NBUF * CHUNK
                pltpu.async_copy(idx_hbm.at[pl.ds(t0n, CHUNK)], idx_b, isems[k])

            pltpu.async_copy(
                rows_b, out_hbm.at[pl.ds(t0 * D, CHUNK * D)], ssems[k]
            )
        return carry

    lax.fori_loop(0, NBODY, body, 0)

    for k in range(NBUF):
        pltpu.make_async_copy(
            rows_v[k], out_hbm.at[pl.ds(base * D, CHUNK * D)], ssems[k]
        ).wait()


def kernel(token_idxs, weight):
    idx = token_idxs.reshape(B)
    out = _emb_lookup(idx, weight.reshape(VOCAB * D))
    return out.reshape(BATCH, HIST, D)


# speedup vs baseline: 1.2810x; 1.2810x over previous
import functools

import jax
import jax.numpy as jnp
from jax import lax
from jax.experimental import pallas as pl
from jax.experimental.pallas import tpu as pltpu
from jax.experimental.pallas import tpu_sc as plsc

VOCAB = 1000
BATCH = 16384
HIST = 200
D = 32
B = BATCH * HIST
NC = 2
NS = 16
NW = NC * NS
TOK_PER_W = B // NW
CHUNK = 1024
NITER = TOK_PER_W // CHUNK
NGRP = CHUNK // 16
NBUF = 2
NBODY = NITER // NBUF

_mesh = plsc.VectorSubcoreMesh(core_axis_name="c", subcore_axis_name="s")


@functools.partial(
    pl.kernel,
    mesh=_mesh,
    compiler_params=pltpu.CompilerParams(needs_layout_passes=False),
    out_type=jax.ShapeDtypeStruct((B * D,), jnp.float32),
    scratch_types=[
        pltpu.VMEM((VOCAB * D,), jnp.float32),
        [pltpu.VMEM((CHUNK,), jnp.int32)] * NBUF,
        [pltpu.VMEM((CHUNK * D,), jnp.float32)] * NBUF,
        [pltpu.SemaphoreType.DMA] * NBUF,
        [pltpu.SemaphoreType.DMA] * NBUF,
    ],
)
def _emb_lookup(idx_hbm, w_hbm, out_hbm, w_loc, idx_v, rows_v, ssems, isems):
    wid = lax.axis_index("s") * NC + lax.axis_index("c")
    base = wid * TOK_PER_W

    pltpu.sync_copy(w_hbm, w_loc)

    lane32 = lax.iota(jnp.int32, 16) * D

    def body(s, carry):
        for k in range(NBUF):
            i = s * NBUF + k
            t0 = base + i * CHUNK
            idx_b = idx_v[k]
            rows_b = rows_v[k]

            @pl.when(s > 0)
            def _drain_prev():
                pltpu.make_async_copy(
                    rows_b, out_hbm.at[pl.ds(t0 * D, CHUNK * D)], ssems[k]
                ).wait()
                pltpu.make_async_copy(
                    idx_hbm.at[pl.ds(t0, CHUNK)], idx_b, isems[k]
                ).wait()

            @pl.when(s == 0)
            def _prime_idx():
                pltpu.sync_copy(idx_hbm.at[pl.ds(t0, CHUNK)], idx_b)

            @plsc.parallel_loop(0, NGRP, unroll=2)
            def _group(g):
                tok = idx_b[pl.ds(g * 16, 16)]
                gbase = tok * D
                obase = lane32 + g * (16 * D)
                for d in range(D):
                    vals = plsc.load_gather(w_loc, [gbase + d])
                    plsc.store_scatter(rows_b, [obase + d], vals)

            @pl.when(s < NBODY - 1)
            def _prefetch_idx():
                t0n = t0 + NBUF * CHUNK
                pltpu.async_copy(idx_hbm.at[pl.ds(t0n, CHUNK)], idx_b, isems[k])

            pltpu.async_copy(
                rows_b, out_hbm.at[pl.ds(t0 * D, CHUNK * D)], ssems[k]
            )
        return carry

    lax.fori_loop(0, NBODY, body, 0)

    for k in range(NBUF):
        pltpu.make_async_copy(
            rows_v[k], out_hbm.at[pl.ds(base * D, CHUNK * D)], ssems[k]
        ).wait()


def kernel(token_idxs, weight):
    idx = token_idxs.reshape(B)
    out = _emb_lookup(idx, weight.reshape(VOCAB * D))
    return out.reshape(BATCH, HIST, D)


# speedup vs baseline: 2.8682x; 2.2390x over previous
import functools

import jax
import jax.numpy as jnp
from jax import lax
from jax.experimental import pallas as pl
from jax.experimental.pallas import tpu as pltpu
from jax.experimental.pallas import tpu_sc as plsc

VOCAB = 1000
BATCH = 16384
HIST = 200
D = 32
B = BATCH * HIST
NC = 2
NS = 16
NW = NC * NS
TOK_PER_W = B // NW
CHUNK = 1024
NITER = TOK_PER_W // CHUNK
NGRP = CHUNK // 16
NBUF = 2
NBODY = NITER // NBUF

_mesh = plsc.VectorSubcoreMesh(core_axis_name="c", subcore_axis_name="s")

_SPLAT_DNUMS = lax.GatherDimensionNumbers(
    offset_dims=(), collapsed_slice_dims=(0,), start_index_map=(0,)
)


def _lane_splat(vec, t):
    idx = jnp.full((16, 1), t, jnp.int32)
    return lax.gather(
        vec,
        idx,
        _SPLAT_DNUMS,
        (1,),
        mode=lax.GatherScatterMode.PROMISE_IN_BOUNDS,
    )


@functools.partial(
    pl.kernel,
    mesh=_mesh,
    compiler_params=pltpu.CompilerParams(needs_layout_passes=False),
    out_type=jax.ShapeDtypeStruct((B * D,), jnp.float32),
    scratch_types=[
        pltpu.VMEM((VOCAB * D,), jnp.float32),
        [pltpu.VMEM((CHUNK,), jnp.int32)] * NBUF,
        [pltpu.VMEM((CHUNK * D,), jnp.float32)] * NBUF,
        [pltpu.SemaphoreType.DMA] * NBUF,
        [pltpu.SemaphoreType.DMA] * NBUF,
    ],
)
def _emb_lookup(idx_hbm, w_hbm, out_hbm, w_loc, idx_v, rows_v, ssems, isems):
    wid = lax.axis_index("s") * NC + lax.axis_index("c")
    base = wid * TOK_PER_W

    pltpu.sync_copy(w_hbm, w_loc)

    lane16 = lax.iota(jnp.int32, 16)

    def body(s, carry):
        for k in range(NBUF):
            i = s * NBUF + k
            t0 = base + i * CHUNK
            idx_b = idx_v[k]
            rows_b = rows_v[k]

            @pl.when(s > 0)
            def _drain_prev():
                pltpu.make_async_copy(
                    rows_b, out_hbm.at[pl.ds(t0 * D, CHUNK * D)], ssems[k]
                ).wait()
                pltpu.make_async_copy(
                    idx_hbm.at[pl.ds(t0, CHUNK)], idx_b, isems[k]
                ).wait()

            @pl.when(s == 0)
            def _prime_idx():
                pltpu.sync_copy(idx_hbm.at[pl.ds(t0, CHUNK)], idx_b)

            @plsc.parallel_loop(0, NGRP, unroll=1)
            def _group(g):
                tok16 = idx_b[pl.ds(g * 16, 16)]
                base16 = tok16 * D
                for t in range(16):
                    bs = _lane_splat(base16, t)
                    a0 = bs + lane16
                    v0 = plsc.load_gather(w_loc, [a0])
                    v1 = plsc.load_gather(w_loc, [a0 + 16])
                    o0 = (g * 16 + t) * D
                    rows_b[pl.ds(o0, 16)] = v0
                    rows_b[pl.ds(o0 + 16, 16)] = v1

            @pl.when(s < NBODY - 1)
            def _prefetch_idx():
                t0n = t0 + NBUF * CHUNK
                pltpu.async_copy(idx_hbm.at[pl.ds(t0n, CHUNK)], idx_b, isems[k])

            pltpu.async_copy(
                rows_b, out_hbm.at[pl.ds(t0 * D, CHUNK * D)], ssems[k]
            )
        return carry

    lax.fori_loop(0, NBODY, body, 0)

    for k in range(NBUF):
        pltpu.make_async_copy(
            rows_v[k], out_hbm.at[pl.ds(base * D, CHUNK * D)], ssems[k]
        ).wait()


def kernel(token_idxs, weight):
    idx = token_idxs.reshape(B)
    out = _emb_lookup(idx, weight.reshape(VOCAB * D))
    return out.reshape(BATCH, HIST, D)


# speedup vs baseline: 2.9796x; 1.0388x over previous
import functools

import jax
import jax.numpy as jnp
from jax import lax
from jax.experimental import pallas as pl
from jax.experimental.pallas import tpu as pltpu
from jax.experimental.pallas import tpu_sc as plsc

VOCAB = 1000
BATCH = 16384
HIST = 200
D = 32
B = BATCH * HIST
NC = 2
NS = 16
NW = NC * NS
TOK_PER_W = B // NW
CHUNK = 1024
NITER = TOK_PER_W // CHUNK
NBUF = 2
NBODY = NITER // NBUF

SROWS = 4
SS = SROWS * 128
CC = CHUNK - SS
NGRP = CC // 16

_mesh = plsc.VectorSubcoreMesh(core_axis_name="c", subcore_axis_name="s")

_SPLAT_DNUMS = lax.GatherDimensionNumbers(
    offset_dims=(), collapsed_slice_dims=(0,), start_index_map=(0,)
)


def _lane_splat(vec, t):
    idx = jnp.full((16, 1), t, jnp.int32)
    return lax.gather(
        vec,
        idx,
        _SPLAT_DNUMS,
        (1,),
        mode=lax.GatherScatterMode.PROMISE_IN_BOUNDS,
    )


@functools.partial(
    pl.kernel,
    mesh=_mesh,
    compiler_params=pltpu.CompilerParams(
        needs_layout_passes=False, use_tc_tiling_on_sc=False
    ),
    out_type=jax.ShapeDtypeStruct((B, D), jnp.float32),
    scratch_types=[
        pltpu.VMEM_SHARED((VOCAB, D), jnp.float32),
        pltpu.VMEM((VOCAB, D), jnp.float32),
        [pltpu.VMEM((CHUNK,), jnp.int32)] * NBUF,
        [pltpu.VMEM((SS, D), jnp.float32)] * NBUF,
        [pltpu.VMEM((CC, D), jnp.float32)] * NBUF,
        pltpu.SemaphoreType.DMA,
        [pltpu.SemaphoreType.DMA] * NBUF,
        [pltpu.SemaphoreType.DMA] * NBUF,
        [pltpu.SemaphoreType.DMA] * NBUF,
    ],
)
def _emb_lookup(
    idx_hbm, w_hbm, out_hbm,
    w_sh, w_loc, idx_v, rows_s, rows_c,
    gsem, ssems, csems, isems,
):
    sid = lax.axis_index("s")
    wid = sid * NC + lax.axis_index("c")
    base = wid * TOK_PER_W

    pltpu.sync_copy(w_hbm, w_loc)

    @pl.when(sid == 0)
    def _stage_table():
        pltpu.sync_copy(w_hbm, w_sh)

    plsc.subcore_barrier()

    lane16 = lax.iota(jnp.int32, 16)

    def body(s, carry):
        for k in range(NBUF):
            i = s * NBUF + k
            t0 = base + i * CHUNK
            idx_b = idx_v[k]
            rs_b = rows_s[k]
            rc_b = rows_c[k]

            @pl.when(s > 0)
            def _drain_prev():
                pltpu.make_async_copy(
                    rs_b, out_hbm.at[pl.ds(t0, SS)], ssems[k]
                ).wait()
                pltpu.make_async_copy(
                    rc_b, out_hbm.at[pl.ds(t0, CC)], csems[k]
                ).wait()
                pltpu.make_async_copy(
                    idx_hbm.at[pl.ds(t0, CHUNK)], idx_b, isems[k]
                ).wait()

            @pl.when(s == 0)
            def _prime_idx():
                pltpu.sync_copy(idx_hbm.at[pl.ds(t0, CHUNK)], idx_b)

            copies = [
                pltpu.async_copy(
                    w_sh.at[idx_b.at[pl.ds(j * 128, 128)]],
                    rs_b.at[pl.ds(j * 128, 128)],
                    gsem,
                )
                for j in range(SROWS)
            ]

            @plsc.parallel_loop(0, NGRP, unroll=1)
            def _group(g):
                tok16 = idx_b[pl.ds(SS + g * 16, 16)]
                for t in range(16):
                    bs = _lane_splat(tok16, t)
                    v0 = plsc.load_gather(w_loc, [bs, lane16])
                    v1 = plsc.load_gather(w_loc, [bs, lane16 + 16])
                    tloc = g * 16 + t
                    rc_b[tloc, pl.ds(0, 16)] = v0
                    rc_b[tloc, pl.ds(16, 16)] = v1

            for cp in copies:
                cp.wait()

            @pl.when(s < NBODY - 1)
            def _prefetch_idx():
                t0n = t0 + NBUF * CHUNK
                pltpu.async_copy(idx_hbm.at[pl.ds(t0n, CHUNK)], idx_b, isems[k])

            pltpu.async_copy(rs_b, out_hbm.at[pl.ds(t0, SS)], ssems[k])
            pltpu.async_copy(rc_b, out_hbm.at[pl.ds(t0 + SS, CC)], csems[k])
        return carry

    lax.fori_loop(0, NBODY, body, 0)

    for k in range(NBUF):
        pltpu.make_async_copy(
            rows_s[k], out_hbm.at[pl.ds(base, SS)], ssems[k]
        ).wait()
        pltpu.make_async_copy(
            rows_c[k], out_hbm.at[pl.ds(base, CC)], csems[k]
        ).wait()


def kernel(token_idxs, weight):
    idx = token_idxs.reshape(B)
    out = _emb_lookup(idx, weight)
    return out.reshape(BATCH, HIST, D)
